# Initial kernel scaffold; baseline (speedup 1.0000x reference)
import jax
import jax.numpy as jnp
from jax.experimental import pallas as pl


def kernel(x, edge_index, batch, W1, b1, W2, b2, W3, b3, g1, be1, g2, be2, g3, be3, fw1, fb1, fw2, fb2):
    raise NotImplementedError("write your pallas kernel here")


# speedup vs baseline: 8.0281x; 8.0281x over previous
import functools

import jax
import jax.numpy as jnp
from jax import lax
from jax.experimental import pallas as pl
from jax.experimental.pallas import tpu as pltpu
from jax.experimental.pallas import tpu_sc as plsc

N = 10000
NG = 64
D = 128
EPS = 1e-5
NB = 79
NP = NB * 128
NW = 32
NPT = NP // 16
E = 320000
CPT = 79
EP = NW * CPT * 128
TRASH = N
DW = 16

_MESH = plsc.VectorSubcoreMesh(core_axis_name="c", subcore_axis_name="s")


def _sc_degree(dsts, zeros128, ones_blk):
    @functools.partial(
        pl.kernel,
        out_type=jax.ShapeDtypeStruct((2, NP, D), jnp.float32),
        mesh=_MESH,
        scratch_types=[
            pltpu.VMEM((CPT, 128), jnp.int32),
            pltpu.VMEM((128, D), jnp.float32),
            pltpu.VMEM_SHARED((NP, D), jnp.float32),
            pltpu.SemaphoreType.DMA,
        ],
    )
    def k(dst_hbm, zeros_hbm, ones_hbm, out_hbm, dst_scr, ones, deg, sem):
        cid = lax.axis_index("c")
        sid = lax.axis_index("s")
        slab = cid * 16 + sid
        pltpu.sync_copy(ones_hbm, ones)
        pltpu.sync_copy(dst_hbm.at[slab], dst_scr)
        base = sid * NPT
        pltpu.sync_copy(zeros_hbm.at[pl.ds(base, NPT)],
                        deg.at[pl.ds(base, NPT)])
        plsc.subcore_barrier()

        def edge_body(j, carry):
            pltpu.sync_copy(ones, deg.at[dst_scr.at[j]], add=True)
            return carry

        lax.fori_loop(0, CPT, edge_body, 0)
        plsc.subcore_barrier()
        pltpu.sync_copy(deg.at[pl.ds(base, NPT)],
                        out_hbm.at[cid, pl.ds(base, NPT)])

    return k(dsts, zeros128, ones_blk)


def _sc_propagate(y, srcs, dsts, zeros128):
    @functools.partial(
        pl.kernel,
        out_type=jax.ShapeDtypeStruct((2, NP, D), jnp.float32),
        mesh=_MESH,
        scratch_types=[
            pltpu.VMEM((CPT, 128), jnp.int32),
            pltpu.VMEM((CPT, 128), jnp.int32),
            pltpu.VMEM((128, D), jnp.float32),
            pltpu.VMEM_SHARED((NP, D), jnp.float32),
            pltpu.SemaphoreType.DMA,
        ],
    )
    def k(y_hbm, src_hbm, dst_hbm, zeros_hbm, out_hbm, src_scr, dst_scr,
          rows, acc, sem):
        cid = lax.axis_index("c")
        sid = lax.axis_index("s")
        slab = cid * 16 + sid
        pltpu.sync_copy(src_hbm.at[slab], src_scr)
        pltpu.sync_copy(dst_hbm.at[slab], dst_scr)
        base = sid * NPT
        pltpu.sync_copy(zeros_hbm.at[pl.ds(base, NPT)],
                        acc.at[pl.ds(base, NPT)])
        plsc.subcore_barrier()

        def edge_body(j, carry):
            pltpu.async_copy(y_hbm.at[src_scr.at[j]], rows, sem).wait()
            pltpu.sync_copy(rows, acc.at[dst_scr.at[j]], add=True)
            return carry

        lax.fori_loop(0, CPT, edge_body, 0)
        plsc.subcore_barrier()
        pltpu.sync_copy(acc.at[pl.ds(base, NPT)],
                        out_hbm.at[cid, pl.ds(base, NPT)])

    return k(y, srcs, dsts, zeros128)


def _tc_dinv(deg_parts):
    def body(dp_ref, o_ref):
        i = pl.program_id(0)
        deg = dp_ref[0] + dp_ref[1] + 1.0
        rows = lax.broadcasted_iota(jnp.int32, (128, 1), 0) + i * 128
        mask = (rows < N).astype(jnp.float32)
        o_ref[...] = lax.rsqrt(deg) * mask

    return pl.pallas_call(
        body,
        grid=(NB,),
        in_specs=[pl.BlockSpec((2, 128, D), lambda i: (0, i, 0))],
        out_specs=pl.BlockSpec((128, D), lambda i: (i, 0)),
        out_shape=jax.ShapeDtypeStruct((NP, D), jnp.float32),
    )(deg_parts)


def _tc_y1(x_p, W, dinv):
    def body(x_ref, w_ref, dv_ref, y_ref):
        y_ref[...] = jnp.dot(x_ref[...], w_ref[...],
                             preferred_element_type=jnp.float32) * dv_ref[...]

    return pl.pallas_call(
        body,
        grid=(NB,),
        in_specs=[
            pl.BlockSpec((128, D), lambda i: (i, 0)),
            pl.BlockSpec((D, D), lambda i: (0, 0)),
            pl.BlockSpec((128, D), lambda i: (i, 0)),
        ],
        out_specs=pl.BlockSpec((128, D), lambda i: (i, 0)),
        out_shape=jax.ShapeDtypeStruct((NP, D), jnp.float32),
    )(x_p, W, dinv)


def _tc_combine(acc, y, dinv, b):
    def body(acc_ref, y_ref, dv_ref, b_ref, z_ref, st_ref):
        i = pl.program_id(0)
        rows = lax.broadcasted_iota(jnp.int32, (128, 1), 0) + i * 128
        mask = (rows < N).astype(jnp.float32)
        a = acc_ref[0] + acc_ref[1]
        z = (dv_ref[...] * (a + y_ref[...]) + b_ref[...]) * mask
        z_ref[...] = z

        @pl.when(i == 0)
        def _():
            st_ref[...] = jnp.zeros((8, 128), jnp.float32)

        st_ref[0:1, :] += jnp.sum(z, axis=0, keepdims=True)
        st_ref[1:2, :] += jnp.sum(z * z, axis=0, keepdims=True)

    return pl.pallas_call(
        body,
        grid=(NB,),
        in_specs=[
            pl.BlockSpec((2, 128, D), lambda i: (0, i, 0)),
            pl.BlockSpec((128, D), lambda i: (i, 0)),
            pl.BlockSpec((128, D), lambda i: (i, 0)),
            pl.BlockSpec((1, D), lambda i: (0, 0)),
        ],
        out_specs=[
            pl.BlockSpec((128, D), lambda i: (i, 0)),
            pl.BlockSpec((8, 128), lambda i: (0, 0)),
        ],
        out_shape=[
            jax.ShapeDtypeStruct((NP, D), jnp.float32),
            jax.ShapeDtypeStruct((8, 128), jnp.float32),
        ],
    )(acc, y, dinv, b)


def _bn_affine(st_ref, g_ref, be_ref):
    mean = st_ref[0:1, :] * (1.0 / N)
    var = st_ref[1:2, :] * (1.0 / N) - mean * mean
    s = g_ref[...] * lax.rsqrt(var + EPS)
    t = be_ref[...] - mean * s
    return s, t


def _tc_norm_mm(z, st, g, be, W, dinv):
    def body(z_ref, st_ref, g_ref, be_ref, w_ref, dv_ref, y_ref):
        s, t = _bn_affine(st_ref, g_ref, be_ref)
        h = jnp.maximum(z_ref[...] * s + t, 0.0)
        y_ref[...] = jnp.dot(h, w_ref[...],
                             preferred_element_type=jnp.float32) * dv_ref[...]

    return pl.pallas_call(
        body,
        grid=(NB,),
        in_specs=[
            pl.BlockSpec((128, D), lambda i: (i, 0)),
            pl.BlockSpec((8, 128), lambda i: (0, 0)),
            pl.BlockSpec((1, D), lambda i: (0, 0)),
            pl.BlockSpec((1, D), lambda i: (0, 0)),
            pl.BlockSpec((D, D), lambda i: (0, 0)),
            pl.BlockSpec((128, D), lambda i: (i, 0)),
        ],
        out_specs=pl.BlockSpec((128, D), lambda i: (i, 0)),
        out_shape=jax.ShapeDtypeStruct((NP, D), jnp.float32),
    )(z, st, g, be, W, dinv)


def _tc_pool(z, st, g, be, batch_p):
    def body(z_ref, st_ref, g_ref, be_ref, b_ref, p_ref, c_ref):
        i = pl.program_id(0)
        s, t = _bn_affine(st_ref, g_ref, be_ref)
        h = jnp.maximum(z_ref[...] * s + t, 0.0)
        bb = jnp.broadcast_to(b_ref[0], (NG, 128))
        gg = lax.broadcasted_iota(jnp.int32, (NG, 128), 0)
        oh = (gg == bb).astype(jnp.float32)

        @pl.when(i == 0)
        def _():
            p_ref[...] = jnp.zeros((NG, D), jnp.float32)
            c_ref[...] = jnp.zeros((NG, D), jnp.float32)

        p_ref[...] += jnp.dot(oh, h, preferred_element_type=jnp.float32)
        c_ref[...] += jnp.broadcast_to(
            jnp.sum(oh, axis=1, keepdims=True), (NG, D))

    return pl.pallas_call(
        body,
        grid=(NB,),
        in_specs=[
            pl.BlockSpec((128, D), lambda i: (i, 0)),
            pl.BlockSpec((8, 128), lambda i: (0, 0)),
            pl.BlockSpec((1, D), lambda i: (0, 0)),
            pl.BlockSpec((1, D), lambda i: (0, 0)),
            pl.BlockSpec((1, 1, 128), lambda i: (i, 0, 0)),
        ],
        out_specs=[
            pl.BlockSpec((NG, D), lambda i: (0, 0)),
            pl.BlockSpec((NG, D), lambda i: (0, 0)),
        ],
        out_shape=[
            jax.ShapeDtypeStruct((NG, D), jnp.float32),
            jax.ShapeDtypeStruct((NG, D), jnp.float32),
        ],
    )(z, st, g, be, batch_p)


def _tc_head(P, cnt, fw1, fb1, fw2p, fb2p):
    def body(p_ref, c_ref, w1_ref, b1_ref, w2_ref, b2_ref, o_ref):
        pooled = p_ref[...] / jnp.maximum(c_ref[...], 1.0)
        h = jnp.maximum(
            jnp.dot(pooled, w1_ref[...],
                    preferred_element_type=jnp.float32) + b1_ref[...], 0.0)
        o_ref[...] = jnp.dot(h, w2_ref[...],
                             preferred_element_type=jnp.float32) + b2_ref[...]

    return pl.pallas_call(
        body,
        out_shape=jax.ShapeDtypeStruct((NG, D), jnp.float32),
    )(P, cnt, fw1, fb1, fw2p, fb2p)


def kernel(x, edge_index, batch, W1, b1, W2, b2, W3, b3, g1, be1, g2, be2,
           g3, be3, fw1, fb1, fw2, fb2):
    src = edge_index[0].astype(jnp.int32)
    dst = edge_index[1].astype(jnp.int32)
    srcs = jnp.concatenate(
        [src, jnp.zeros((EP - E,), jnp.int32)]).reshape(NW, CPT, 128)
    dsts = jnp.concatenate(
        [dst, jnp.full((EP - E,), TRASH, jnp.int32)]).reshape(NW, CPT, 128)
    x_p = jnp.pad(x, ((0, NP - N), (0, 0)))
    batch_p = jnp.pad(batch.astype(jnp.int32), (0, NP - N),
                      constant_values=NG).reshape(NB, 1, 128)
    b1r, b2r, b3r = b1.reshape(1, D), b2.reshape(1, D), b3.reshape(1, D)
    g1r, g2r, g3r = g1.reshape(1, D), g2.reshape(1, D), g3.reshape(1, D)
    be1r, be2r, be3r = be1.reshape(1, D), be2.reshape(1, D), be3.reshape(1, D)
    fb1r = fb1.reshape(1, D)
    fw2p = jnp.pad(fw2, ((0, 0), (0, D - fw2.shape[1])))
    fb2p = jnp.pad(fb2, (0, D - fb2.shape[0])).reshape(1, D)

    zeros128 = jnp.zeros((NP, D), jnp.float32)
    ones_blk = jnp.ones((128, D), jnp.float32)
    deg_parts = _sc_degree(dsts, zeros128, ones_blk)
    dinv = _tc_dinv(deg_parts)

    y1 = _tc_y1(x_p, W1, dinv)
    acc1 = _sc_propagate(y1, srcs, dsts, zeros128)
    z1, st1 = _tc_combine(acc1, y1, dinv, b1r)

    y2 = _tc_norm_mm(z1, st1, g1r, be1r, W2, dinv)
    acc2 = _sc_propagate(y2, srcs, dsts, zeros128)
    z2, st2 = _tc_combine(acc2, y2, dinv, b2r)

    y3 = _tc_norm_mm(z2, st2, g2r, be2r, W3, dinv)
    acc3 = _sc_propagate(y3, srcs, dsts, zeros128)
    z3, st3 = _tc_combine(acc3, y3, dinv, b3r)

    P, cnt = _tc_pool(z3, st3, g3r, be3r, batch_p)
    out = _tc_head(P, cnt, fw1, fb1r, fw2p, fb2p)
    return out[:, :16]


# speedup vs baseline: 9.1656x; 1.1417x over previous
import functools

import jax
import jax.numpy as jnp
from jax import lax
from jax.experimental import pallas as pl
from jax.experimental.pallas import tpu as pltpu
from jax.experimental.pallas import tpu_sc as plsc

N = 10000
NG = 64
D = 128
EPS = 1e-5
NB = 79
NP = NB * 128
NW = 32
NPT = NP // 16
E = 320000
CPT = 79
EP = NW * CPT * 128
TRASH = N
DW = 16

_MESH = plsc.VectorSubcoreMesh(core_axis_name="c", subcore_axis_name="s")


def _sc_degree(dsts, zeros128, ones_blk):
    @functools.partial(
        pl.kernel,
        out_type=jax.ShapeDtypeStruct((2, NP, D), jnp.float32),
        mesh=_MESH,
        scratch_types=[
            pltpu.VMEM((CPT, 128), jnp.int32),
            pltpu.VMEM((128, D), jnp.float32),
            pltpu.VMEM_SHARED((NP, D), jnp.float32),
            pltpu.SemaphoreType.DMA,
        ],
    )
    def k(dst_hbm, zeros_hbm, ones_hbm, out_hbm, dst_scr, ones, deg, sem):
        cid = lax.axis_index("c")
        sid = lax.axis_index("s")
        slab = cid * 16 + sid
        pltpu.sync_copy(ones_hbm, ones)
        pltpu.sync_copy(dst_hbm.at[slab], dst_scr)
        base = sid * NPT
        pltpu.sync_copy(zeros_hbm.at[pl.ds(base, NPT)],
                        deg.at[pl.ds(base, NPT)])
        plsc.subcore_barrier()

        def edge_body(j, carry):
            pltpu.sync_copy(ones, deg.at[dst_scr.at[j]], add=True)
            return carry

        lax.fori_loop(0, CPT, edge_body, 0)
        plsc.subcore_barrier()
        pltpu.sync_copy(deg.at[pl.ds(base, NPT)],
                        out_hbm.at[cid, pl.ds(base, NPT)])

    return k(dsts, zeros128, ones_blk)


def _sc_propagate(y, srcs, dsts, zeros128):
    @functools.partial(
        pl.kernel,
        out_type=jax.ShapeDtypeStruct((2, NP, D), jnp.float32),
        mesh=_MESH,
        scratch_types=[
            pltpu.VMEM((4, 128), jnp.int32),
            pltpu.VMEM((4, 128), jnp.int32),
            pltpu.VMEM((2, 128, D), jnp.float32),
            pltpu.VMEM_SHARED((NP, D), jnp.float32),
            pltpu.SemaphoreType.DMA((4,)),
            pltpu.SemaphoreType.DMA((2,)),
        ],
    )
    def k(y_hbm, src_hbm, dst_hbm, zeros_hbm, out_hbm, src_scr, dst_scr,
          rows, acc, isem, gsem):
        cid = lax.axis_index("c")
        sid = lax.axis_index("s")
        slab = cid * 16 + sid
        base = sid * NPT
        pltpu.sync_copy(zeros_hbm.at[pl.ds(base, NPT)],
                        acc.at[pl.ds(base, NPT)])

        def start_idx(m, slot):
            pltpu.async_copy(src_hbm.at[slab, m], src_scr.at[slot],
                             isem.at[slot])
            pltpu.async_copy(dst_hbm.at[slab, m], dst_scr.at[slot],
                             isem.at[slot])

        def wait_idx(m, slot):
            pltpu.make_async_copy(src_hbm.at[slab, m], src_scr.at[slot],
                                  isem.at[slot]).wait()
            pltpu.make_async_copy(dst_hbm.at[slab, m], dst_scr.at[slot],
                                  isem.at[slot]).wait()

        for m in range(3):
            start_idx(m, m)
        plsc.subcore_barrier()
        wait_idx(0, 0)
        pltpu.async_copy(y_hbm.at[src_scr.at[0]], rows.at[0], gsem.at[0])

        def edge_body(j, carry):
            b = lax.rem(j, 2)
            nb = 1 - b
            s1 = lax.rem(j + 1, 4)
            s3 = lax.rem(j + 3, 4)

            @pl.when(j + 1 < CPT)
            def _():
                wait_idx(j + 1, s1)
                pltpu.async_copy(y_hbm.at[src_scr.at[s1]], rows.at[nb],
                                 gsem.at[nb])

            @pl.when(j + 3 < CPT)
            def _():
                start_idx(j + 3, s3)

            pltpu.make_async_copy(y_hbm.at[src_scr.at[lax.rem(j, 4)]],
                                  rows.at[b], gsem.at[b]).wait()
            pltpu.sync_copy(rows.at[b], acc.at[dst_scr.at[lax.rem(j, 4)]],
                            add=True)
            return carry

        lax.fori_loop(0, CPT, edge_body, 0)
        plsc.subcore_barrier()
        pltpu.sync_copy(acc.at[pl.ds(base, NPT)],
                        out_hbm.at[cid, pl.ds(base, NPT)])

    return k(y, srcs, dsts, zeros128)


def _tc_dinv(deg_parts):
    def body(dp_ref, o_ref):
        i = pl.program_id(0)
        deg = dp_ref[0] + dp_ref[1] + 1.0
        rows = lax.broadcasted_iota(jnp.int32, (128, 1), 0) + i * 128
        mask = (rows < N).astype(jnp.float32)
        o_ref[...] = lax.rsqrt(deg) * mask

    return pl.pallas_call(
        body,
        grid=(NB,),
        in_specs=[pl.BlockSpec((2, 128, D), lambda i: (0, i, 0))],
        out_specs=pl.BlockSpec((128, D), lambda i: (i, 0)),
        out_shape=jax.ShapeDtypeStruct((NP, D), jnp.float32),
    )(deg_parts)


def _tc_y1(x_p, W, dinv):
    def body(x_ref, w_ref, dv_ref, y_ref):
        y_ref[...] = jnp.dot(x_ref[...], w_ref[...],
                             preferred_element_type=jnp.float32) * dv_ref[...]

    return pl.pallas_call(
        body,
        grid=(NB,),
        in_specs=[
            pl.BlockSpec((128, D), lambda i: (i, 0)),
            pl.BlockSpec((D, D), lambda i: (0, 0)),
            pl.BlockSpec((128, D), lambda i: (i, 0)),
        ],
        out_specs=pl.BlockSpec((128, D), lambda i: (i, 0)),
        out_shape=jax.ShapeDtypeStruct((NP, D), jnp.float32),
    )(x_p, W, dinv)


def _tc_combine(acc, y, dinv, b):
    def body(acc_ref, y_ref, dv_ref, b_ref, z_ref, st_ref):
        i = pl.program_id(0)
        rows = lax.broadcasted_iota(jnp.int32, (128, 1), 0) + i * 128
        mask = (rows < N).astype(jnp.float32)
        a = acc_ref[0] + acc_ref[1]
        z = (dv_ref[...] * (a + y_ref[...]) + b_ref[...]) * mask
        z_ref[...] = z

        @pl.when(i == 0)
        def _():
            st_ref[...] = jnp.zeros((8, 128), jnp.float32)

        st_ref[0:1, :] += jnp.sum(z, axis=0, keepdims=True)
        st_ref[1:2, :] += jnp.sum(z * z, axis=0, keepdims=True)

    return pl.pallas_call(
        body,
        grid=(NB,),
        in_specs=[
            pl.BlockSpec((2, 128, D), lambda i: (0, i, 0)),
            pl.BlockSpec((128, D), lambda i: (i, 0)),
            pl.BlockSpec((128, D), lambda i: (i, 0)),
            pl.BlockSpec((1, D), lambda i: (0, 0)),
        ],
        out_specs=[
            pl.BlockSpec((128, D), lambda i: (i, 0)),
            pl.BlockSpec((8, 128), lambda i: (0, 0)),
        ],
        out_shape=[
            jax.ShapeDtypeStruct((NP, D), jnp.float32),
            jax.ShapeDtypeStruct((8, 128), jnp.float32),
        ],
    )(acc, y, dinv, b)


def _bn_affine(st_ref, g_ref, be_ref):
    mean = st_ref[0:1, :] * (1.0 / N)
    var = st_ref[1:2, :] * (1.0 / N) - mean * mean
    s = g_ref[...] * lax.rsqrt(var + EPS)
    t = be_ref[...] - mean * s
    return s, t


def _tc_norm_mm(z, st, g, be, W, dinv):
    def body(z_ref, st_ref, g_ref, be_ref, w_ref, dv_ref, y_ref):
        s, t = _bn_affine(st_ref, g_ref, be_ref)
        h = jnp.maximum(z_ref[...] * s + t, 0.0)
        y_ref[...] = jnp.dot(h, w_ref[...],
                             preferred_element_type=jnp.float32) * dv_ref[...]

    return pl.pallas_call(
        body,
        grid=(NB,),
        in_specs=[
            pl.BlockSpec((128, D), lambda i: (i, 0)),
            pl.BlockSpec((8, 128), lambda i: (0, 0)),
            pl.BlockSpec((1, D), lambda i: (0, 0)),
            pl.BlockSpec((1, D), lambda i: (0, 0)),
            pl.BlockSpec((D, D), lambda i: (0, 0)),
            pl.BlockSpec((128, D), lambda i: (i, 0)),
        ],
        out_specs=pl.BlockSpec((128, D), lambda i: (i, 0)),
        out_shape=jax.ShapeDtypeStruct((NP, D), jnp.float32),
    )(z, st, g, be, W, dinv)


def _tc_pool(z, st, g, be, batch_p):
    def body(z_ref, st_ref, g_ref, be_ref, b_ref, p_ref, c_ref):
        i = pl.program_id(0)
        s, t = _bn_affine(st_ref, g_ref, be_ref)
        h = jnp.maximum(z_ref[...] * s + t, 0.0)
        bb = jnp.broadcast_to(b_ref[0], (NG, 128))
        gg = lax.broadcasted_iota(jnp.int32, (NG, 128), 0)
        oh = (gg == bb).astype(jnp.float32)

        @pl.when(i == 0)
        def _():
            p_ref[...] = jnp.zeros((NG, D), jnp.float32)
            c_ref[...] = jnp.zeros((NG, D), jnp.float32)

        p_ref[...] += jnp.dot(oh, h, preferred_element_type=jnp.float32)
        c_ref[...] += jnp.broadcast_to(
            jnp.sum(oh, axis=1, keepdims=True), (NG, D))

    return pl.pallas_call(
        body,
        grid=(NB,),
        in_specs=[
            pl.BlockSpec((128, D), lambda i: (i, 0)),
            pl.BlockSpec((8, 128), lambda i: (0, 0)),
            pl.BlockSpec((1, D), lambda i: (0, 0)),
            pl.BlockSpec((1, D), lambda i: (0, 0)),
            pl.BlockSpec((1, 1, 128), lambda i: (i, 0, 0)),
        ],
        out_specs=[
            pl.BlockSpec((NG, D), lambda i: (0, 0)),
            pl.BlockSpec((NG, D), lambda i: (0, 0)),
        ],
        out_shape=[
            jax.ShapeDtypeStruct((NG, D), jnp.float32),
            jax.ShapeDtypeStruct((NG, D), jnp.float32),
        ],
    )(z, st, g, be, batch_p)


def _tc_head(P, cnt, fw1, fb1, fw2p, fb2p):
    def body(p_ref, c_ref, w1_ref, b1_ref, w2_ref, b2_ref, o_ref):
        pooled = p_ref[...] / jnp.maximum(c_ref[...], 1.0)
        h = jnp.maximum(
            jnp.dot(pooled, w1_ref[...],
                    preferred_element_type=jnp.float32) + b1_ref[...], 0.0)
        o_ref[...] = jnp.dot(h, w2_ref[...],
                             preferred_element_type=jnp.float32) + b2_ref[...]

    return pl.pallas_call(
        body,
        out_shape=jax.ShapeDtypeStruct((NG, D), jnp.float32),
    )(P, cnt, fw1, fb1, fw2p, fb2p)


def kernel(x, edge_index, batch, W1, b1, W2, b2, W3, b3, g1, be1, g2, be2,
           g3, be3, fw1, fb1, fw2, fb2):
    src = edge_index[0].astype(jnp.int32)
    dst = edge_index[1].astype(jnp.int32)
    srcs = jnp.concatenate(
        [src, jnp.zeros((EP - E,), jnp.int32)]).reshape(NW, CPT, 128)
    dsts = jnp.concatenate(
        [dst, jnp.full((EP - E,), TRASH, jnp.int32)]).reshape(NW, CPT, 128)
    x_p = jnp.pad(x, ((0, NP - N), (0, 0)))
    batch_p = jnp.pad(batch.astype(jnp.int32), (0, NP - N),
                      constant_values=NG).reshape(NB, 1, 128)
    b1r, b2r, b3r = b1.reshape(1, D), b2.reshape(1, D), b3.reshape(1, D)
    g1r, g2r, g3r = g1.reshape(1, D), g2.reshape(1, D), g3.reshape(1, D)
    be1r, be2r, be3r = be1.reshape(1, D), be2.reshape(1, D), be3.reshape(1, D)
    fb1r = fb1.reshape(1, D)
    fw2p = jnp.pad(fw2, ((0, 0), (0, D - fw2.shape[1])))
    fb2p = jnp.pad(fb2, (0, D - fb2.shape[0])).reshape(1, D)

    zeros128 = jnp.zeros((NP, D), jnp.float32)
    ones_blk = jnp.ones((128, D), jnp.float32)
    deg_parts = _sc_degree(dsts, zeros128, ones_blk)
    dinv = _tc_dinv(deg_parts)

    y1 = _tc_y1(x_p, W1, dinv)
    acc1 = _sc_propagate(y1, srcs, dsts, zeros128)
    z1, st1 = _tc_combine(acc1, y1, dinv, b1r)

    y2 = _tc_norm_mm(z1, st1, g1r, be1r, W2, dinv)
    acc2 = _sc_propagate(y2, srcs, dsts, zeros128)
    z2, st2 = _tc_combine(acc2, y2, dinv, b2r)

    y3 = _tc_norm_mm(z2, st2, g2r, be2r, W3, dinv)
    acc3 = _sc_propagate(y3, srcs, dsts, zeros128)
    z3, st3 = _tc_combine(acc3, y3, dinv, b3r)

    P, cnt = _tc_pool(z3, st3, g3r, be3r, batch_p)
    out = _tc_head(P, cnt, fw1, fb1r, fw2p, fb2p)
    return out[:, :16]


# speedup vs baseline: 9.1788x; 1.0014x over previous
import functools

import jax
import jax.numpy as jnp
from jax import lax
from jax.experimental import pallas as pl
from jax.experimental.pallas import tpu as pltpu
from jax.experimental.pallas import tpu_sc as plsc

N = 10000
NG = 64
D = 128
EPS = 1e-5
NB = 79
NP = NB * 128
NW = 32
NPT = NP // 16
E = 320000
CPT = 79
EP = NW * CPT * 128
TRASH = N
DW = 16

_MESH = plsc.VectorSubcoreMesh(core_axis_name="c", subcore_axis_name="s")


def _sc_degree(dsts, zeros128, ones_blk):
    @functools.partial(
        pl.kernel,
        out_type=jax.ShapeDtypeStruct((2, NP, D), jnp.float32),
        mesh=_MESH,
        scratch_types=[
            pltpu.VMEM((CPT, 128), jnp.int32),
            pltpu.VMEM((128, D), jnp.float32),
            pltpu.VMEM_SHARED((NP, D), jnp.float32),
            pltpu.SemaphoreType.DMA,
        ],
    )
    def k(dst_hbm, zeros_hbm, ones_hbm, out_hbm, dst_scr, ones, deg, sem):
        cid = lax.axis_index("c")
        sid = lax.axis_index("s")
        slab = cid * 16 + sid
        pltpu.sync_copy(ones_hbm, ones)
        pltpu.sync_copy(dst_hbm.at[slab], dst_scr)
        base = sid * NPT
        pltpu.sync_copy(zeros_hbm.at[pl.ds(base, NPT)],
                        deg.at[pl.ds(base, NPT)])
        plsc.subcore_barrier()

        def edge_body(j, carry):
            pltpu.sync_copy(ones, deg.at[dst_scr.at[j]], add=True)
            return carry

        lax.fori_loop(0, CPT, edge_body, 0)
        plsc.subcore_barrier()
        pltpu.sync_copy(deg.at[pl.ds(base, NPT)],
                        out_hbm.at[cid, pl.ds(base, NPT)])

    return k(dsts, zeros128, ones_blk)


def _sc_propagate(y, srcs, dsts, zeros128):
    @functools.partial(
        pl.kernel,
        out_type=jax.ShapeDtypeStruct((2, NP, D), jnp.float32),
        mesh=_MESH,
        scratch_types=[
            pltpu.VMEM((4, 128), jnp.int32),
            pltpu.VMEM((4, 128), jnp.int32),
            pltpu.VMEM((2, 128, D), jnp.float32),
            pltpu.VMEM_SHARED((NP, D), jnp.float32),
            pltpu.SemaphoreType.DMA((4,)),
            pltpu.SemaphoreType.DMA((2,)),
            pltpu.SemaphoreType.DMA((2,)),
        ],
    )
    def k(y_hbm, src_hbm, dst_hbm, zeros_hbm, out_hbm, src_scr, dst_scr,
          rows, acc, isem, gsem, ssem):
        cid = lax.axis_index("c")
        sid = lax.axis_index("s")
        slab = cid * 16 + sid
        base = sid * NPT
        pltpu.sync_copy(zeros_hbm.at[pl.ds(base, NPT)],
                        acc.at[pl.ds(base, NPT)])

        def start_idx(m, slot):
            pltpu.async_copy(src_hbm.at[slab, m], src_scr.at[slot],
                             isem.at[slot])
            pltpu.async_copy(dst_hbm.at[slab, m], dst_scr.at[slot],
                             isem.at[slot])

        def wait_idx(m, slot):
            pltpu.make_async_copy(src_hbm.at[slab, m], src_scr.at[slot],
                                  isem.at[slot]).wait()
            pltpu.make_async_copy(dst_hbm.at[slab, m], dst_scr.at[slot],
                                  isem.at[slot]).wait()

        for m in range(3):
            start_idx(m, m)
        plsc.subcore_barrier()
        wait_idx(0, 0)
        pltpu.async_copy(y_hbm.at[src_scr.at[0]], rows.at[0], gsem.at[0])

        def edge_body(j, carry):
            b = lax.rem(j, 2)
            nb = 1 - b
            s1 = lax.rem(j + 1, 4)
            s3 = lax.rem(j + 3, 4)

            @pl.when(j + 1 < CPT)
            def _():
                wait_idx(j + 1, s1)

                @pl.when(j >= 1)
                def _():
                    pltpu.make_async_copy(
                        rows.at[nb], acc.at[dst_scr.at[lax.rem(j + 3, 4)]],
                        ssem.at[nb]).wait()

                pltpu.async_copy(y_hbm.at[src_scr.at[s1]], rows.at[nb],
                                 gsem.at[nb])

            @pl.when(j + 3 < CPT)
            def _():
                start_idx(j + 3, s3)

            pltpu.make_async_copy(y_hbm.at[src_scr.at[lax.rem(j, 4)]],
                                  rows.at[b], gsem.at[b]).wait()
            pltpu.async_copy(rows.at[b], acc.at[dst_scr.at[lax.rem(j, 4)]],
                             ssem.at[b], add=True)
            return carry

        lax.fori_loop(0, CPT, edge_body, 0)
        for jt in (CPT - 2, CPT - 1):
            pltpu.make_async_copy(rows.at[jt % 2],
                                  acc.at[dst_scr.at[jt % 4]],
                                  ssem.at[jt % 2]).wait()
        plsc.subcore_barrier()
        pltpu.sync_copy(acc.at[pl.ds(base, NPT)],
                        out_hbm.at[cid, pl.ds(base, NPT)])

    return k(y, srcs, dsts, zeros128)


def _tc_dinv(deg_parts):
    def body(dp_ref, o_ref):
        i = pl.program_id(0)
        deg = dp_ref[0] + dp_ref[1] + 1.0
        rows = lax.broadcasted_iota(jnp.int32, (128, 1), 0) + i * 128
        mask = (rows < N).astype(jnp.float32)
        o_ref[...] = lax.rsqrt(deg) * mask

    return pl.pallas_call(
        body,
        grid=(NB,),
        in_specs=[pl.BlockSpec((2, 128, D), lambda i: (0, i, 0))],
        out_specs=pl.BlockSpec((128, D), lambda i: (i, 0)),
        out_shape=jax.ShapeDtypeStruct((NP, D), jnp.float32),
    )(deg_parts)


def _tc_y1(x_p, W, dinv):
    def body(x_ref, w_ref, dv_ref, y_ref):
        y_ref[...] = jnp.dot(x_ref[...], w_ref[...],
                             preferred_element_type=jnp.float32) * dv_ref[...]

    return pl.pallas_call(
        body,
        grid=(NB,),
        in_specs=[
            pl.BlockSpec((128, D), lambda i: (i, 0)),
            pl.BlockSpec((D, D), lambda i: (0, 0)),
            pl.BlockSpec((128, D), lambda i: (i, 0)),
        ],
        out_specs=pl.BlockSpec((128, D), lambda i: (i, 0)),
        out_shape=jax.ShapeDtypeStruct((NP, D), jnp.float32),
    )(x_p, W, dinv)


def _tc_combine(acc, y, dinv, b):
    def body(acc_ref, y_ref, dv_ref, b_ref, z_ref, st_ref):
        i = pl.program_id(0)
        rows = lax.broadcasted_iota(jnp.int32, (128, 1), 0) + i * 128
        mask = (rows < N).astype(jnp.float32)
        a = acc_ref[0] + acc_ref[1]
        z = (dv_ref[...] * (a + y_ref[...]) + b_ref[...]) * mask
        z_ref[...] = z

        @pl.when(i == 0)
        def _():
            st_ref[...] = jnp.zeros((8, 128), jnp.float32)

        st_ref[0:1, :] += jnp.sum(z, axis=0, keepdims=True)
        st_ref[1:2, :] += jnp.sum(z * z, axis=0, keepdims=True)

    return pl.pallas_call(
        body,
        grid=(NB,),
        in_specs=[
            pl.BlockSpec((2, 128, D), lambda i: (0, i, 0)),
            pl.BlockSpec((128, D), lambda i: (i, 0)),
            pl.BlockSpec((128, D), lambda i: (i, 0)),
            pl.BlockSpec((1, D), lambda i: (0, 0)),
        ],
        out_specs=[
            pl.BlockSpec((128, D), lambda i: (i, 0)),
            pl.BlockSpec((8, 128), lambda i: (0, 0)),
        ],
        out_shape=[
            jax.ShapeDtypeStruct((NP, D), jnp.float32),
            jax.ShapeDtypeStruct((8, 128), jnp.float32),
        ],
    )(acc, y, dinv, b)


def _bn_affine(st_ref, g_ref, be_ref):
    mean = st_ref[0:1, :] * (1.0 / N)
    var = st_ref[1:2, :] * (1.0 / N) - mean * mean
    s = g_ref[...] * lax.rsqrt(var + EPS)
    t = be_ref[...] - mean * s
    return s, t


def _tc_norm_mm(z, st, g, be, W, dinv):
    def body(z_ref, st_ref, g_ref, be_ref, w_ref, dv_ref, y_ref):
        s, t = _bn_affine(st_ref, g_ref, be_ref)
        h = jnp.maximum(z_ref[...] * s + t, 0.0)
        y_ref[...] = jnp.dot(h, w_ref[...],
                             preferred_element_type=jnp.float32) * dv_ref[...]

    return pl.pallas_call(
        body,
        grid=(NB,),
        in_specs=[
            pl.BlockSpec((128, D), lambda i: (i, 0)),
            pl.BlockSpec((8, 128), lambda i: (0, 0)),
            pl.BlockSpec((1, D), lambda i: (0, 0)),
            pl.BlockSpec((1, D), lambda i: (0, 0)),
            pl.BlockSpec((D, D), lambda i: (0, 0)),
            pl.BlockSpec((128, D), lambda i: (i, 0)),
        ],
        out_specs=pl.BlockSpec((128, D), lambda i: (i, 0)),
        out_shape=jax.ShapeDtypeStruct((NP, D), jnp.float32),
    )(z, st, g, be, W, dinv)


def _tc_pool(z, st, g, be, batch_p):
    def body(z_ref, st_ref, g_ref, be_ref, b_ref, p_ref, c_ref):
        i = pl.program_id(0)
        s, t = _bn_affine(st_ref, g_ref, be_ref)
        h = jnp.maximum(z_ref[...] * s + t, 0.0)
        bb = jnp.broadcast_to(b_ref[0], (NG, 128))
        gg = lax.broadcasted_iota(jnp.int32, (NG, 128), 0)
        oh = (gg == bb).astype(jnp.float32)

        @pl.when(i == 0)
        def _():
            p_ref[...] = jnp.zeros((NG, D), jnp.float32)
            c_ref[...] = jnp.zeros((NG, D), jnp.float32)

        p_ref[...] += jnp.dot(oh, h, preferred_element_type=jnp.float32)
        c_ref[...] += jnp.broadcast_to(
            jnp.sum(oh, axis=1, keepdims=True), (NG, D))

    return pl.pallas_call(
        body,
        grid=(NB,),
        in_specs=[
            pl.BlockSpec((128, D), lambda i: (i, 0)),
            pl.BlockSpec((8, 128), lambda i: (0, 0)),
            pl.BlockSpec((1, D), lambda i: (0, 0)),
            pl.BlockSpec((1, D), lambda i: (0, 0)),
            pl.BlockSpec((1, 1, 128), lambda i: (i, 0, 0)),
        ],
        out_specs=[
            pl.BlockSpec((NG, D), lambda i: (0, 0)),
            pl.BlockSpec((NG, D), lambda i: (0, 0)),
        ],
        out_shape=[
            jax.ShapeDtypeStruct((NG, D), jnp.float32),
            jax.ShapeDtypeStruct((NG, D), jnp.float32),
        ],
    )(z, st, g, be, batch_p)


def _tc_head(P, cnt, fw1, fb1, fw2p, fb2p):
    def body(p_ref, c_ref, w1_ref, b1_ref, w2_ref, b2_ref, o_ref):
        pooled = p_ref[...] / jnp.maximum(c_ref[...], 1.0)
        h = jnp.maximum(
            jnp.dot(pooled, w1_ref[...],
                    preferred_element_type=jnp.float32) + b1_ref[...], 0.0)
        o_ref[...] = jnp.dot(h, w2_ref[...],
                             preferred_element_type=jnp.float32) + b2_ref[...]

    return pl.pallas_call(
        body,
        out_shape=jax.ShapeDtypeStruct((NG, D), jnp.float32),
    )(P, cnt, fw1, fb1, fw2p, fb2p)


def kernel(x, edge_index, batch, W1, b1, W2, b2, W3, b3, g1, be1, g2, be2,
           g3, be3, fw1, fb1, fw2, fb2):
    src = edge_index[0].astype(jnp.int32)
    dst = edge_index[1].astype(jnp.int32)
    srcs = jnp.concatenate(
        [src, jnp.zeros((EP - E,), jnp.int32)]).reshape(NW, CPT, 128)
    dsts = jnp.concatenate(
        [dst, jnp.full((EP - E,), TRASH, jnp.int32)]).reshape(NW, CPT, 128)
    x_p = jnp.pad(x, ((0, NP - N), (0, 0)))
    batch_p = jnp.pad(batch.astype(jnp.int32), (0, NP - N),
                      constant_values=NG).reshape(NB, 1, 128)
    b1r, b2r, b3r = b1.reshape(1, D), b2.reshape(1, D), b3.reshape(1, D)
    g1r, g2r, g3r = g1.reshape(1, D), g2.reshape(1, D), g3.reshape(1, D)
    be1r, be2r, be3r = be1.reshape(1, D), be2.reshape(1, D), be3.reshape(1, D)
    fb1r = fb1.reshape(1, D)
    fw2p = jnp.pad(fw2, ((0, 0), (0, D - fw2.shape[1])))
    fb2p = jnp.pad(fb2, (0, D - fb2.shape[0])).reshape(1, D)

    zeros128 = jnp.zeros((NP, D), jnp.float32)
    ones_blk = jnp.ones((128, D), jnp.float32)
    deg_parts = _sc_degree(dsts, zeros128, ones_blk)
    dinv = _tc_dinv(deg_parts)

    y1 = _tc_y1(x_p, W1, dinv)
    acc1 = _sc_propagate(y1, srcs, dsts, zeros128)
    z1, st1 = _tc_combine(acc1, y1, dinv, b1r)

    y2 = _tc_norm_mm(z1, st1, g1r, be1r, W2, dinv)
    acc2 = _sc_propagate(y2, srcs, dsts, zeros128)
    z2, st2 = _tc_combine(acc2, y2, dinv, b2r)

    y3 = _tc_norm_mm(z2, st2, g2r, be2r, W3, dinv)
    acc3 = _sc_propagate(y3, srcs, dsts, zeros128)
    z3, st3 = _tc_combine(acc3, y3, dinv, b3r)

    P, cnt = _tc_pool(z3, st3, g3r, be3r, batch_p)
    out = _tc_head(P, cnt, fw1, fb1r, fw2p, fb2p)
    return out[:, :16]


# speedup vs baseline: 9.4377x; 1.0282x over previous
import functools

import jax
import jax.numpy as jnp
from jax import lax
from jax.experimental import pallas as pl
from jax.experimental.pallas import tpu as pltpu
from jax.experimental.pallas import tpu_sc as plsc

N = 10000
NG = 64
D = 128
EPS = 1e-5
NB = 79
NP = NB * 128
NW = 32
NPT = NP // 16
E = 320000
CPT = 79
EP = NW * CPT * 128
CPT_A = 44
CPT_B = 114
CPT_MAX = max(CPT_A, CPT_B)
EPA = 16 * CPT_A * 128
EPP = 16 * (CPT_A + CPT_B) * 128
TRASH = N
DW = 16

_MESH = plsc.VectorSubcoreMesh(core_axis_name="c", subcore_axis_name="s")


def _sc_degree(dsts, zeros128, ones_blk):
    @functools.partial(
        pl.kernel,
        out_type=jax.ShapeDtypeStruct((2, NP, D), jnp.float32),
        mesh=_MESH,
        scratch_types=[
            pltpu.VMEM((CPT, 128), jnp.int32),
            pltpu.VMEM((128, D), jnp.float32),
            pltpu.VMEM_SHARED((NP, D), jnp.float32),
            pltpu.SemaphoreType.DMA,
        ],
    )
    def k(dst_hbm, zeros_hbm, ones_hbm, out_hbm, dst_scr, ones, deg, sem):
        cid = lax.axis_index("c")
        sid = lax.axis_index("s")
        slab = cid * 16 + sid
        pltpu.sync_copy(ones_hbm, ones)
        pltpu.sync_copy(dst_hbm.at[slab], dst_scr)
        base = sid * NPT
        pltpu.sync_copy(zeros_hbm.at[pl.ds(base, NPT)],
                        deg.at[pl.ds(base, NPT)])
        plsc.subcore_barrier()

        def edge_body(j, carry):
            pltpu.sync_copy(ones, deg.at[dst_scr.at[j]], add=True)
            return carry

        lax.fori_loop(0, CPT, edge_body, 0)
        plsc.subcore_barrier()
        pltpu.sync_copy(deg.at[pl.ds(base, NPT)],
                        out_hbm.at[cid, pl.ds(base, NPT)])

    return k(dsts, zeros128, ones_blk)


def _sc_propagate(y, srcs, dsts, zeros128):
    @functools.partial(
        pl.kernel,
        out_type=jax.ShapeDtypeStruct((2, NP, D), jnp.float32),
        mesh=_MESH,
        scratch_types=[
            pltpu.VMEM((4, 128), jnp.int32),
            pltpu.VMEM((4, 128), jnp.int32),
            pltpu.VMEM((2, 128, D), jnp.float32),
            pltpu.VMEM_SHARED((NP, D), jnp.float32),
            pltpu.SemaphoreType.DMA((4,)),
            pltpu.SemaphoreType.DMA((2,)),
            pltpu.SemaphoreType.DMA((2,)),
        ],
    )
    def k(y_hbm, src_hbm, dst_hbm, zeros_hbm, out_hbm, src_scr, dst_scr,
          rows, acc, isem, gsem, ssem):
        cid = lax.axis_index("c")
        sid = lax.axis_index("s")
        mycpt = jnp.where(cid == 0, CPT_A, CPT_B)
        qbase = jnp.where(cid == 0, sid * CPT_A,
                          16 * CPT_A + sid * CPT_B)
        base = sid * NPT
        pltpu.sync_copy(zeros_hbm.at[pl.ds(base, NPT)],
                        acc.at[pl.ds(base, NPT)])

        def start_idx(m, slot):
            pltpu.async_copy(src_hbm.at[qbase + m], src_scr.at[slot],
                             isem.at[slot])
            pltpu.async_copy(dst_hbm.at[qbase + m], dst_scr.at[slot],
                             isem.at[slot])

        def wait_idx(m, slot):
            pltpu.make_async_copy(src_hbm.at[qbase + m], src_scr.at[slot],
                                  isem.at[slot]).wait()
            pltpu.make_async_copy(dst_hbm.at[qbase + m], dst_scr.at[slot],
                                  isem.at[slot]).wait()

        for m in range(3):
            start_idx(m, m)
        plsc.subcore_barrier()
        wait_idx(0, 0)
        pltpu.async_copy(y_hbm.at[src_scr.at[0]], rows.at[0], gsem.at[0])

        def edge_body(j, carry):
            b = lax.rem(j, 2)
            nb = 1 - b
            s1 = lax.rem(j + 1, 4)
            s3 = lax.rem(j + 3, 4)

            @pl.when(j + 1 < mycpt)
            def _():
                wait_idx(j + 1, s1)

                @pl.when(j >= 1)
                def _():
                    pltpu.make_async_copy(
                        rows.at[nb], acc.at[dst_scr.at[lax.rem(j + 3, 4)]],
                        ssem.at[nb]).wait()

                pltpu.async_copy(y_hbm.at[src_scr.at[s1]], rows.at[nb],
                                 gsem.at[nb])

            @pl.when(j + 3 < mycpt)
            def _():
                start_idx(j + 3, s3)

            @pl.when(j < mycpt)
            def _():
                pltpu.make_async_copy(y_hbm.at[src_scr.at[lax.rem(j, 4)]],
                                      rows.at[b], gsem.at[b]).wait()
                pltpu.async_copy(rows.at[b],
                                 acc.at[dst_scr.at[lax.rem(j, 4)]],
                                 ssem.at[b], add=True)

            return carry

        lax.fori_loop(0, CPT_MAX, edge_body, 0)
        for kk in (2, 1):
            jt = mycpt - kk
            pltpu.make_async_copy(rows.at[lax.rem(jt, 2)],
                                  acc.at[dst_scr.at[lax.rem(jt, 4)]],
                                  ssem.at[lax.rem(jt, 2)]).wait()
        plsc.subcore_barrier()
        pltpu.sync_copy(acc.at[pl.ds(base, NPT)],
                        out_hbm.at[cid, pl.ds(base, NPT)])

    return k(y, srcs, dsts, zeros128)


def _tc_dinv(deg_parts):
    def body(dp_ref, o_ref):
        i = pl.program_id(0)
        deg = dp_ref[0] + dp_ref[1] + 1.0
        rows = lax.broadcasted_iota(jnp.int32, (128, 1), 0) + i * 128
        mask = (rows < N).astype(jnp.float32)
        o_ref[...] = lax.rsqrt(deg) * mask

    return pl.pallas_call(
        body,
        grid=(NB,),
        in_specs=[pl.BlockSpec((2, 128, D), lambda i: (0, i, 0))],
        out_specs=pl.BlockSpec((128, D), lambda i: (i, 0)),
        out_shape=jax.ShapeDtypeStruct((NP, D), jnp.float32),
    )(deg_parts)


def _tc_y1(x_p, W, dinv):
    def body(x_ref, w_ref, dv_ref, y_ref):
        y_ref[...] = jnp.dot(x_ref[...], w_ref[...],
                             preferred_element_type=jnp.float32) * dv_ref[...]

    return pl.pallas_call(
        body,
        grid=(NB,),
        in_specs=[
            pl.BlockSpec((128, D), lambda i: (i, 0)),
            pl.BlockSpec((D, D), lambda i: (0, 0)),
            pl.BlockSpec((128, D), lambda i: (i, 0)),
        ],
        out_specs=pl.BlockSpec((128, D), lambda i: (i, 0)),
        out_shape=jax.ShapeDtypeStruct((NP, D), jnp.float32),
    )(x_p, W, dinv)


def _tc_combine(acc, y, dinv, b):
    def body(acc_ref, y_ref, dv_ref, b_ref, z_ref, st_ref):
        i = pl.program_id(0)
        rows = lax.broadcasted_iota(jnp.int32, (128, 1), 0) + i * 128
        mask = (rows < N).astype(jnp.float32)
        a = acc_ref[0] + acc_ref[1]
        z = (dv_ref[...] * (a + y_ref[...]) + b_ref[...]) * mask
        z_ref[...] = z

        @pl.when(i == 0)
        def _():
            st_ref[...] = jnp.zeros((8, 128), jnp.float32)

        st_ref[0:1, :] += jnp.sum(z, axis=0, keepdims=True)
        st_ref[1:2, :] += jnp.sum(z * z, axis=0, keepdims=True)

    return pl.pallas_call(
        body,
        grid=(NB,),
        in_specs=[
            pl.BlockSpec((2, 128, D), lambda i: (0, i, 0)),
            pl.BlockSpec((128, D), lambda i: (i, 0)),
            pl.BlockSpec((128, D), lambda i: (i, 0)),
            pl.BlockSpec((1, D), lambda i: (0, 0)),
        ],
        out_specs=[
            pl.BlockSpec((128, D), lambda i: (i, 0)),
            pl.BlockSpec((8, 128), lambda i: (0, 0)),
        ],
        out_shape=[
            jax.ShapeDtypeStruct((NP, D), jnp.float32),
            jax.ShapeDtypeStruct((8, 128), jnp.float32),
        ],
    )(acc, y, dinv, b)


def _bn_affine(st_ref, g_ref, be_ref):
    mean = st_ref[0:1, :] * (1.0 / N)
    var = st_ref[1:2, :] * (1.0 / N) - mean * mean
    s = g_ref[...] * lax.rsqrt(var + EPS)
    t = be_ref[...] - mean * s
    return s, t


def _tc_norm_mm(z, st, g, be, W, dinv):
    def body(z_ref, st_ref, g_ref, be_ref, w_ref, dv_ref, y_ref):
        s, t = _bn_affine(st_ref, g_ref, be_ref)
        h = jnp.maximum(z_ref[...] * s + t, 0.0)
        y_ref[...] = jnp.dot(h, w_ref[...],
                             preferred_element_type=jnp.float32) * dv_ref[...]

    return pl.pallas_call(
        body,
        grid=(NB,),
        in_specs=[
            pl.BlockSpec((128, D), lambda i: (i, 0)),
            pl.BlockSpec((8, 128), lambda i: (0, 0)),
            pl.BlockSpec((1, D), lambda i: (0, 0)),
            pl.BlockSpec((1, D), lambda i: (0, 0)),
            pl.BlockSpec((D, D), lambda i: (0, 0)),
            pl.BlockSpec((128, D), lambda i: (i, 0)),
        ],
        out_specs=pl.BlockSpec((128, D), lambda i: (i, 0)),
        out_shape=jax.ShapeDtypeStruct((NP, D), jnp.float32),
    )(z, st, g, be, W, dinv)


def _tc_pool(z, st, g, be, batch_p):
    def body(z_ref, st_ref, g_ref, be_ref, b_ref, p_ref, c_ref):
        i = pl.program_id(0)
        s, t = _bn_affine(st_ref, g_ref, be_ref)
        h = jnp.maximum(z_ref[...] * s + t, 0.0)
        bb = jnp.broadcast_to(b_ref[0], (NG, 128))
        gg = lax.broadcasted_iota(jnp.int32, (NG, 128), 0)
        oh = (gg == bb).astype(jnp.float32)

        @pl.when(i == 0)
        def _():
            p_ref[...] = jnp.zeros((NG, D), jnp.float32)
            c_ref[...] = jnp.zeros((NG, D), jnp.float32)

        p_ref[...] += jnp.dot(oh, h, preferred_element_type=jnp.float32)
        c_ref[...] += jnp.broadcast_to(
            jnp.sum(oh, axis=1, keepdims=True), (NG, D))

    return pl.pallas_call(
        body,
        grid=(NB,),
        in_specs=[
            pl.BlockSpec((128, D), lambda i: (i, 0)),
            pl.BlockSpec((8, 128), lambda i: (0, 0)),
            pl.BlockSpec((1, D), lambda i: (0, 0)),
            pl.BlockSpec((1, D), lambda i: (0, 0)),
            pl.BlockSpec((1, 1, 128), lambda i: (i, 0, 0)),
        ],
        out_specs=[
            pl.BlockSpec((NG, D), lambda i: (0, 0)),
            pl.BlockSpec((NG, D), lambda i: (0, 0)),
        ],
        out_shape=[
            jax.ShapeDtypeStruct((NG, D), jnp.float32),
            jax.ShapeDtypeStruct((NG, D), jnp.float32),
        ],
    )(z, st, g, be, batch_p)


def _tc_head(P, cnt, fw1, fb1, fw2p, fb2p):
    def body(p_ref, c_ref, w1_ref, b1_ref, w2_ref, b2_ref, o_ref):
        pooled = p_ref[...] / jnp.maximum(c_ref[...], 1.0)
        h = jnp.maximum(
            jnp.dot(pooled, w1_ref[...],
                    preferred_element_type=jnp.float32) + b1_ref[...], 0.0)
        o_ref[...] = jnp.dot(h, w2_ref[...],
                             preferred_element_type=jnp.float32) + b2_ref[...]

    return pl.pallas_call(
        body,
        out_shape=jax.ShapeDtypeStruct((NG, D), jnp.float32),
    )(P, cnt, fw1, fb1, fw2p, fb2p)


def kernel(x, edge_index, batch, W1, b1, W2, b2, W3, b3, g1, be1, g2, be2,
           g3, be3, fw1, fb1, fw2, fb2):
    src = edge_index[0].astype(jnp.int32)
    dst = edge_index[1].astype(jnp.int32)
    flat_src = jnp.concatenate([src, jnp.zeros((EP - E,), jnp.int32)])
    flat_dst = jnp.concatenate([dst, jnp.full((EP - E,), TRASH, jnp.int32)])
    dsts = flat_dst.reshape(NW, CPT, 128)
    srcf = flat_src.reshape(NW * CPT, 128)
    dstf = flat_dst.reshape(NW * CPT, 128)
    x_p = jnp.pad(x, ((0, NP - N), (0, 0)))
    batch_p = jnp.pad(batch.astype(jnp.int32), (0, NP - N),
                      constant_values=NG).reshape(NB, 1, 128)
    b1r, b2r, b3r = b1.reshape(1, D), b2.reshape(1, D), b3.reshape(1, D)
    g1r, g2r, g3r = g1.reshape(1, D), g2.reshape(1, D), g3.reshape(1, D)
    be1r, be2r, be3r = be1.reshape(1, D), be2.reshape(1, D), be3.reshape(1, D)
    fb1r = fb1.reshape(1, D)
    fw2p = jnp.pad(fw2, ((0, 0), (0, D - fw2.shape[1])))
    fb2p = jnp.pad(fb2, (0, D - fb2.shape[0])).reshape(1, D)

    zeros128 = jnp.zeros((NP, D), jnp.float32)
    ones_blk = jnp.ones((128, D), jnp.float32)
    deg_parts = _sc_degree(dsts, zeros128, ones_blk)
    dinv = _tc_dinv(deg_parts)

    y1 = _tc_y1(x_p, W1, dinv)
    acc1 = _sc_propagate(y1, srcf, dstf, zeros128)
    z1, st1 = _tc_combine(acc1, y1, dinv, b1r)

    y2 = _tc_norm_mm(z1, st1, g1r, be1r, W2, dinv)
    acc2 = _sc_propagate(y2, srcf, dstf, zeros128)
    z2, st2 = _tc_combine(acc2, y2, dinv, b2r)

    y3 = _tc_norm_mm(z2, st2, g2r, be2r, W3, dinv)
    acc3 = _sc_propagate(y3, srcf, dstf, zeros128)
    z3, st3 = _tc_combine(acc3, y3, dinv, b3r)

    P, cnt = _tc_pool(z3, st3, g3r, be3r, batch_p)
    out = _tc_head(P, cnt, fw1, fb1r, fw2p, fb2p)
    return out[:, :16]


# speedup vs baseline: 9.6079x; 1.0180x over previous
import functools

import jax
import jax.numpy as jnp
from jax import lax
from jax.experimental import pallas as pl
from jax.experimental.pallas import tpu as pltpu
from jax.experimental.pallas import tpu_sc as plsc

N = 10000
NG = 64
D = 128
EPS = 1e-5
NB = 79
NP = NB * 128
NW = 32
NPT = NP // 16
E = 320000
CPT = 79
EP = NW * CPT * 128
CPT_A = 44
CPT_B = 114
CPT_MAX = max(CPT_A, CPT_B)
EPA = 16 * CPT_A * 128
EPP = 16 * (CPT_A + CPT_B) * 128
TRASH = N
DW = 16

_MESH = plsc.VectorSubcoreMesh(core_axis_name="c", subcore_axis_name="s")


def _sc_degree(dsts, zeros128, ones_blk):
    @functools.partial(
        pl.kernel,
        out_type=jax.ShapeDtypeStruct((2, NP, D), jnp.float32),
        mesh=_MESH,
        scratch_types=[
            pltpu.VMEM((CPT, 128), jnp.int32),
            pltpu.VMEM((128, D), jnp.float32),
            pltpu.VMEM_SHARED((NP, D), jnp.float32),
            pltpu.SemaphoreType.DMA,
        ],
    )
    def k(dst_hbm, zeros_hbm, ones_hbm, out_hbm, dst_scr, ones, deg, sem):
        cid = lax.axis_index("c")
        sid = lax.axis_index("s")
        slab = cid * 16 + sid
        pltpu.sync_copy(ones_hbm, ones)
        pltpu.sync_copy(dst_hbm.at[slab], dst_scr)
        base = sid * NPT
        pltpu.sync_copy(zeros_hbm.at[pl.ds(base, NPT)],
                        deg.at[pl.ds(base, NPT)])
        plsc.subcore_barrier()

        def edge_body(j, carry):
            pltpu.sync_copy(ones, deg.at[dst_scr.at[j]], add=True)
            return carry

        lax.fori_loop(0, CPT, edge_body, 0)
        plsc.subcore_barrier()
        pltpu.sync_copy(deg.at[pl.ds(base, NPT)],
                        out_hbm.at[cid, pl.ds(base, NPT)])

    return k(dsts, zeros128, ones_blk)


def _sc_propagate(y, srcs, dsts, zeros128):
    @functools.partial(
        pl.kernel,
        out_type=jax.ShapeDtypeStruct((2, NP, D), jnp.float32),
        mesh=_MESH,
        scratch_types=[
            pltpu.VMEM((4, 128), jnp.int32),
            pltpu.VMEM((4, 128), jnp.int32),
            pltpu.VMEM((2, 128, D), jnp.float32),
            pltpu.VMEM_SHARED((NP, D), jnp.float32),
            pltpu.SemaphoreType.DMA((4,)),
            pltpu.SemaphoreType.DMA((2,)),
            pltpu.SemaphoreType.DMA((2,)),
        ],
    )
    def k(y_hbm, src_hbm, dst_hbm, zeros_hbm, out_hbm, src_scr, dst_scr,
          rows, acc, isem, gsem, ssem):
        cid = lax.axis_index("c")
        sid = lax.axis_index("s")
        mycpt = jnp.where(cid == 0, CPT_A, CPT_B)
        qbase = jnp.where(cid == 0, sid * CPT_A,
                          16 * CPT_A + sid * CPT_B)
        base = sid * NPT
        pltpu.sync_copy(zeros_hbm.at[pl.ds(base, NPT)],
                        acc.at[pl.ds(base, NPT)])

        def start_idx(m, slot):
            pltpu.async_copy(src_hbm.at[qbase + m], src_scr.at[slot],
                             isem.at[slot])
            pltpu.async_copy(dst_hbm.at[qbase + m], dst_scr.at[slot],
                             isem.at[slot])

        def wait_idx(m, slot):
            pltpu.make_async_copy(src_hbm.at[qbase + m], src_scr.at[slot],
                                  isem.at[slot]).wait()
            pltpu.make_async_copy(dst_hbm.at[qbase + m], dst_scr.at[slot],
                                  isem.at[slot]).wait()

        for m in range(3):
            start_idx(m, m)
        plsc.subcore_barrier()
        wait_idx(0, 0)
        pltpu.async_copy(y_hbm.at[src_scr.at[0]], rows.at[0], gsem.at[0])

        def edge_body(j, carry):
            b = lax.rem(j, 2)
            nb = 1 - b
            s1 = lax.rem(j + 1, 4)
            s3 = lax.rem(j + 3, 4)

            @pl.when(j + 1 < mycpt)
            def _():
                wait_idx(j + 1, s1)

                @pl.when(j >= 1)
                def _():
                    pltpu.make_async_copy(
                        rows.at[nb], acc.at[dst_scr.at[lax.rem(j + 3, 4)]],
                        ssem.at[nb]).wait()

                pltpu.async_copy(y_hbm.at[src_scr.at[s1]], rows.at[nb],
                                 gsem.at[nb])

            @pl.when(j + 3 < mycpt)
            def _():
                start_idx(j + 3, s3)

            @pl.when(j < mycpt)
            def _():
                pltpu.make_async_copy(y_hbm.at[src_scr.at[lax.rem(j, 4)]],
                                      rows.at[b], gsem.at[b]).wait()
                pltpu.async_copy(rows.at[b],
                                 acc.at[dst_scr.at[lax.rem(j, 4)]],
                                 ssem.at[b], add=True)

            return carry

        lax.fori_loop(0, CPT_MAX, edge_body, 0)
        for kk in (2, 1):
            jt = mycpt - kk
            pltpu.make_async_copy(rows.at[lax.rem(jt, 2)],
                                  acc.at[dst_scr.at[lax.rem(jt, 4)]],
                                  ssem.at[lax.rem(jt, 2)]).wait()
        plsc.subcore_barrier()
        pltpu.sync_copy(acc.at[pl.ds(base, NPT)],
                        out_hbm.at[cid, pl.ds(base, NPT)])

    return k(y, srcs, dsts, zeros128)


def _tc_y1_dinv(x_p, W, deg_parts):
    def body(x_ref, w_ref, dp_ref, y_ref, dv_ref):
        i = pl.program_id(0)
        deg = dp_ref[0] + dp_ref[1] + 1.0
        rows = lax.broadcasted_iota(jnp.int32, (128, 1), 0) + i * 128
        mask = (rows < N).astype(jnp.float32)
        dinv = lax.rsqrt(deg) * mask
        dv_ref[...] = dinv
        y_ref[...] = jnp.dot(x_ref[...], w_ref[...],
                             preferred_element_type=jnp.float32) * dinv

    return pl.pallas_call(
        body,
        grid=(NB,),
        in_specs=[
            pl.BlockSpec((128, D), lambda i: (i, 0)),
            pl.BlockSpec((D, D), lambda i: (0, 0)),
            pl.BlockSpec((2, 128, D), lambda i: (0, i, 0)),
        ],
        out_specs=[
            pl.BlockSpec((128, D), lambda i: (i, 0)),
            pl.BlockSpec((128, D), lambda i: (i, 0)),
        ],
        out_shape=[
            jax.ShapeDtypeStruct((NP, D), jnp.float32),
            jax.ShapeDtypeStruct((NP, D), jnp.float32),
        ],
    )(x_p, W, deg_parts)


def _tc_combine(acc, y, dinv, b):
    def body(acc_ref, y_ref, dv_ref, b_ref, z_ref, st_ref):
        i = pl.program_id(0)
        rows = lax.broadcasted_iota(jnp.int32, (128, 1), 0) + i * 128
        mask = (rows < N).astype(jnp.float32)
        a = acc_ref[0] + acc_ref[1]
        z = (dv_ref[...] * (a + y_ref[...]) + b_ref[...]) * mask
        z_ref[...] = z

        @pl.when(i == 0)
        def _():
            st_ref[...] = jnp.zeros((8, 128), jnp.float32)

        st_ref[0:1, :] += jnp.sum(z, axis=0, keepdims=True)
        st_ref[1:2, :] += jnp.sum(z * z, axis=0, keepdims=True)

    return pl.pallas_call(
        body,
        grid=(NB,),
        in_specs=[
            pl.BlockSpec((2, 128, D), lambda i: (0, i, 0)),
            pl.BlockSpec((128, D), lambda i: (i, 0)),
            pl.BlockSpec((128, D), lambda i: (i, 0)),
            pl.BlockSpec((1, D), lambda i: (0, 0)),
        ],
        out_specs=[
            pl.BlockSpec((128, D), lambda i: (i, 0)),
            pl.BlockSpec((8, 128), lambda i: (0, 0)),
        ],
        out_shape=[
            jax.ShapeDtypeStruct((NP, D), jnp.float32),
            jax.ShapeDtypeStruct((8, 128), jnp.float32),
        ],
    )(acc, y, dinv, b)


def _bn_affine(st_ref, g_ref, be_ref):
    mean = st_ref[0:1, :] * (1.0 / N)
    var = st_ref[1:2, :] * (1.0 / N) - mean * mean
    s = g_ref[...] * lax.rsqrt(var + EPS)
    t = be_ref[...] - mean * s
    return s, t


def _tc_norm_mm(z, st, g, be, W, dinv):
    def body(z_ref, st_ref, g_ref, be_ref, w_ref, dv_ref, y_ref):
        s, t = _bn_affine(st_ref, g_ref, be_ref)
        h = jnp.maximum(z_ref[...] * s + t, 0.0)
        y_ref[...] = jnp.dot(h, w_ref[...],
                             preferred_element_type=jnp.float32) * dv_ref[...]

    return pl.pallas_call(
        body,
        grid=(NB,),
        in_specs=[
            pl.BlockSpec((128, D), lambda i: (i, 0)),
            pl.BlockSpec((8, 128), lambda i: (0, 0)),
            pl.BlockSpec((1, D), lambda i: (0, 0)),
            pl.BlockSpec((1, D), lambda i: (0, 0)),
            pl.BlockSpec((D, D), lambda i: (0, 0)),
            pl.BlockSpec((128, D), lambda i: (i, 0)),
        ],
        out_specs=pl.BlockSpec((128, D), lambda i: (i, 0)),
        out_shape=jax.ShapeDtypeStruct((NP, D), jnp.float32),
    )(z, st, g, be, W, dinv)


def _tc_pool_head(z, st, g, be, batch_p, fw1, fb1, fw2p, fb2p):
    def body(z_ref, st_ref, g_ref, be_ref, b_ref, w1_ref, b1_ref, w2_ref,
             b2_ref, o_ref, p_ref, c_ref):
        i = pl.program_id(0)
        s, t = _bn_affine(st_ref, g_ref, be_ref)
        h = jnp.maximum(z_ref[...] * s + t, 0.0)
        bb = jnp.broadcast_to(b_ref[0], (NG, 128))
        gg = lax.broadcasted_iota(jnp.int32, (NG, 128), 0)
        oh = (gg == bb).astype(jnp.float32)

        @pl.when(i == 0)
        def _():
            p_ref[...] = jnp.zeros((NG, D), jnp.float32)
            c_ref[...] = jnp.zeros((NG, D), jnp.float32)

        p_ref[...] += jnp.dot(oh, h, preferred_element_type=jnp.float32)
        c_ref[...] += jnp.broadcast_to(
            jnp.sum(oh, axis=1, keepdims=True), (NG, D))

        @pl.when(i == NB - 1)
        def _():
            pooled = p_ref[...] / jnp.maximum(c_ref[...], 1.0)
            hh = jnp.maximum(
                jnp.dot(pooled, w1_ref[...],
                        preferred_element_type=jnp.float32) + b1_ref[...],
                0.0)
            o_ref[...] = jnp.dot(hh, w2_ref[...],
                                 preferred_element_type=jnp.float32) + b2_ref[...]

    return pl.pallas_call(
        body,
        grid=(NB,),
        in_specs=[
            pl.BlockSpec((128, D), lambda i: (i, 0)),
            pl.BlockSpec((8, 128), lambda i: (0, 0)),
            pl.BlockSpec((1, D), lambda i: (0, 0)),
            pl.BlockSpec((1, D), lambda i: (0, 0)),
            pl.BlockSpec((1, 1, 128), lambda i: (i, 0, 0)),
            pl.BlockSpec((D, D), lambda i: (0, 0)),
            pl.BlockSpec((1, D), lambda i: (0, 0)),
            pl.BlockSpec((D, D), lambda i: (0, 0)),
            pl.BlockSpec((1, D), lambda i: (0, 0)),
        ],
        out_specs=pl.BlockSpec((NG, D), lambda i: (0, 0)),
        out_shape=jax.ShapeDtypeStruct((NG, D), jnp.float32),
        scratch_shapes=[
            pltpu.VMEM((NG, D), jnp.float32),
            pltpu.VMEM((NG, D), jnp.float32),
        ],
    )(z, st, g, be, batch_p, fw1, fb1, fw2p, fb2p)


def kernel(x, edge_index, batch, W1, b1, W2, b2, W3, b3, g1, be1, g2, be2,
           g3, be3, fw1, fb1, fw2, fb2):
    src = edge_index[0].astype(jnp.int32)
    dst = edge_index[1].astype(jnp.int32)
    flat_src = jnp.concatenate([src, jnp.zeros((EP - E,), jnp.int32)])
    flat_dst = jnp.concatenate([dst, jnp.full((EP - E,), TRASH, jnp.int32)])
    dsts = flat_dst.reshape(NW, CPT, 128)
    srcf = flat_src.reshape(NW * CPT, 128)
    dstf = flat_dst.reshape(NW * CPT, 128)
    x_p = jnp.pad(x, ((0, NP - N), (0, 0)))
    batch_p = jnp.pad(batch.astype(jnp.int32), (0, NP - N),
                      constant_values=NG).reshape(NB, 1, 128)
    b1r, b2r, b3r = b1.reshape(1, D), b2.reshape(1, D), b3.reshape(1, D)
    g1r, g2r, g3r = g1.reshape(1, D), g2.reshape(1, D), g3.reshape(1, D)
    be1r, be2r, be3r = be1.reshape(1, D), be2.reshape(1, D), be3.reshape(1, D)
    fb1r = fb1.reshape(1, D)
    fw2p = jnp.pad(fw2, ((0, 0), (0, D - fw2.shape[1])))
    fb2p = jnp.pad(fb2, (0, D - fb2.shape[0])).reshape(1, D)

    zeros128 = jnp.zeros((NP, D), jnp.float32)
    ones_blk = jnp.ones((128, D), jnp.float32)
    deg_parts = _sc_degree(dsts, zeros128, ones_blk)

    y1, dinv = _tc_y1_dinv(x_p, W1, deg_parts)
    acc1 = _sc_propagate(y1, srcf, dstf, zeros128)
    z1, st1 = _tc_combine(acc1, y1, dinv, b1r)

    y2 = _tc_norm_mm(z1, st1, g1r, be1r, W2, dinv)
    acc2 = _sc_propagate(y2, srcf, dstf, zeros128)
    z2, st2 = _tc_combine(acc2, y2, dinv, b2r)

    y3 = _tc_norm_mm(z2, st2, g2r, be2r, W3, dinv)
    acc3 = _sc_propagate(y3, srcf, dstf, zeros128)
    z3, st3 = _tc_combine(acc3, y3, dinv, b3r)

    out = _tc_pool_head(z3, st3, g3r, be3r, batch_p, fw1, fb1r, fw2p, fb2p)
    return out[:, :16]
